# baseline (speedup 1.0000x reference)
import jax
import jax.numpy as jnp
from jax.experimental import pallas as pl

HID = 64
NOBJ = 14
TOK = 9


def _dot(x, w):
    return jax.lax.dot_general(
        x, w, (((1,), (0,)), ((), ())), preferred_element_type=jnp.float32,
        precision=jax.lax.Precision.HIGHEST,
    )


def _bd(w):
    z = jnp.zeros_like(w)
    return jnp.concatenate(
        [jnp.concatenate([w, z], axis=1), jnp.concatenate([z, w], axis=1)],
        axis=0)


def _mp_kernel(tt_ref, v_ref, we1_ref, be1_ref, we2_ref, be2_ref,
               wh1_ref, bh1_ref, wh2_ref, bh2_ref, out_ref):
    f32 = jnp.float32
    tt = tt_ref[0, :]
    oh = (jax.lax.broadcasted_iota(jnp.int32, (NOBJ, TOK), 1)
          == tt[:, None]).astype(f32)
    we1 = we1_ref[:]
    Si = oh @ we1[0:TOK, :]
    Sj = oh @ we1[TOK:2 * TOK, :]
    WA = we1[19:23, :] + we1[51:55, :]
    WB = we1[35:39, :] - we1[51:55, :]
    wh1 = wh1_ref[:]
    Tn = oh @ wh1[0:TOK, :]
    Wm = wh1[TOK:TOK + HID, :]
    WV = wh1[TOK + HID + 1:TOK + HID + 5, :]
    be1 = be1_ref[:]
    be2 = be2_ref[:]
    bh1 = bh1_ref[:]
    bh2 = bh2_ref[:]
    we2 = we2_ref[:]
    wh2 = wh2_ref[:]

    WA2 = _bd(WA)
    WB2 = jnp.concatenate([WB, WB], axis=1)
    Wem2 = _bd(_dot(we2, Wm))
    WV2 = _bd(WV)
    Wh2_2 = _bd(wh2)
    Sib = Si + be1
    Tnb = Tn + bh1 + float(NOBJ) * _dot(be2, Wm)

    v = v_ref[:]
    Bs = [_dot(v[:, 4 * c:4 * (c + 1)], WB2)
          + jnp.concatenate([Sj[c:c + 1, :]] * 2, axis=1)
          for c in range(NOBJ)]
    cols = []
    for p in range(NOBJ // 2):
        a0, a1 = 2 * p, 2 * p + 1
        vs2 = v[:, 8 * p:8 * p + 8]
        row = jnp.concatenate([Sib[a0:a0 + 1, :], Sib[a1:a1 + 1, :]], axis=1)
        A2 = _dot(vs2, WA2) + row
        terms = [jnp.maximum(A2 + Bs[c], 0.0) for c in range(NOBJ)]
        while len(terms) > 1:
            terms = [terms[i] + terms[i + 1] for i in range(0, len(terms) - 1, 2)] \
                + ([terms[-1]] if len(terms) % 2 else [])
        acc = terms[0]
        rowt = jnp.concatenate([Tnb[a0:a0 + 1, :], Tnb[a1:a1 + 1, :]], axis=1)
        z = jnp.maximum(_dot(acc, Wem2) + _dot(vs2, WV2) + rowt, 0.0)
        cols.append(_dot(z, Wh2_2) + bh2)
    out_ref[:, :] = jnp.concatenate(cols, axis=1)


def kernel(inputs, type_token, global_token, W_e1, b_e1, W_e2, b_e2,
           W_h1, b_h1, W_h2, b_h2):
    nproc, B, n, d = inputs.shape
    v56 = inputs.reshape(B, n * d)
    BB = 2048
    out = pl.pallas_call(
        _mp_kernel,
        grid=(B // BB,),
        in_specs=[
            pl.BlockSpec((1, n), lambda i: (0, 0)),
            pl.BlockSpec((BB, n * d), lambda i: (i, 0)),
            pl.BlockSpec(W_e1.shape, lambda i: (0, 0)),
            pl.BlockSpec((1, HID), lambda i: (0, 0)),
            pl.BlockSpec(W_e2.shape, lambda i: (0, 0)),
            pl.BlockSpec((1, HID), lambda i: (0, 0)),
            pl.BlockSpec(W_h1.shape, lambda i: (0, 0)),
            pl.BlockSpec((1, HID), lambda i: (0, 0)),
            pl.BlockSpec(W_h2.shape, lambda i: (0, 0)),
            pl.BlockSpec((1, 1), lambda i: (0, 0)),
        ],
        out_specs=pl.BlockSpec((BB, n), lambda i: (i, 0)),
        out_shape=jax.ShapeDtypeStruct((B, n), jnp.float32),
    )(type_token.astype(jnp.int32), v56, W_e1, b_e1.reshape(1, -1),
      W_e2, b_e2.reshape(1, -1), W_h1, b_h1.reshape(1, -1),
      W_h2, b_h2.reshape(1, 1))
    return out.reshape(1, B * n, 1)


# speedup vs baseline: 1.0825x; 1.0825x over previous
import jax
import jax.numpy as jnp
from jax.experimental import pallas as pl

HID = 64
NOBJ = 14
TOK = 9


def _dot(x, w):
    return jax.lax.dot_general(
        x, w, (((1,), (0,)), ((), ())), preferred_element_type=jnp.float32,
        precision=jax.lax.Precision.HIGHEST,
    )


def _bd(w):
    z = jnp.zeros_like(w)
    return jnp.concatenate(
        [jnp.concatenate([w, z], axis=1), jnp.concatenate([z, w], axis=1)],
        axis=0)


def _mp_kernel(tt_ref, v_ref, we1_ref, be1_ref, we2_ref, be2_ref,
               wh1_ref, bh1_ref, wh2_ref, bh2_ref, out_ref):
    f32 = jnp.float32
    tt = tt_ref[0, :]
    oh = (jax.lax.broadcasted_iota(jnp.int32, (NOBJ, TOK), 1)
          == tt[:, None]).astype(f32)
    we1 = we1_ref[:]
    Si = oh @ we1[0:TOK, :]
    Sj = oh @ we1[TOK:2 * TOK, :]
    WA = we1[19:23, :] + we1[51:55, :]
    WB = we1[35:39, :] - we1[51:55, :]
    wh1 = wh1_ref[:]
    Tn = oh @ wh1[0:TOK, :]
    Wm = wh1[TOK:TOK + HID, :]
    WV = wh1[TOK + HID + 1:TOK + HID + 5, :]
    be1 = be1_ref[:]
    be2 = be2_ref[:]
    bh1 = bh1_ref[:]
    bh2 = bh2_ref[:]
    we2 = we2_ref[:]
    wh2 = wh2_ref[:]

    WA2 = _bd(WA)
    WB2 = jnp.concatenate([WB, WB], axis=1)
    Wem2 = _bd(_dot(we2, Wm))
    WV2 = _bd(WV)
    Wh2_2 = _bd(wh2)
    Sib = Si + be1
    Tnb = Tn + bh1 + float(NOBJ) * _dot(be2, Wm)

    v = v_ref[:]
    Bs = [_dot(v[:, 4 * c:4 * (c + 1)], WB2)
          + jnp.concatenate([Sj[c:c + 1, :]] * 2, axis=1)
          for c in range(NOBJ)]
    cols = []
    for p in range(NOBJ // 2):
        a0, a1 = 2 * p, 2 * p + 1
        vs2 = v[:, 8 * p:8 * p + 8]
        row = jnp.concatenate([Sib[a0:a0 + 1, :], Sib[a1:a1 + 1, :]], axis=1)
        A2 = _dot(vs2, WA2) + row
        terms = [jnp.maximum(A2 + Bs[c], 0.0) for c in range(NOBJ)]
        while len(terms) > 1:
            terms = [terms[i] + terms[i + 1] for i in range(0, len(terms) - 1, 2)] \
                + ([terms[-1]] if len(terms) % 2 else [])
        acc = terms[0]
        rowt = jnp.concatenate([Tnb[a0:a0 + 1, :], Tnb[a1:a1 + 1, :]], axis=1)
        z = jnp.maximum(_dot(acc, Wem2) + _dot(vs2, WV2) + rowt, 0.0)
        cols.append(_dot(z, Wh2_2) + bh2)
    out_ref[:, :] = jnp.concatenate(cols, axis=1)


def kernel(inputs, type_token, global_token, W_e1, b_e1, W_e2, b_e2,
           W_h1, b_h1, W_h2, b_h2):
    nproc, B, n, d = inputs.shape
    v56 = inputs.reshape(B, n * d)
    BB = 256
    out = pl.pallas_call(
        _mp_kernel,
        grid=(B // BB,),
        in_specs=[
            pl.BlockSpec((1, n), lambda i: (0, 0)),
            pl.BlockSpec((BB, n * d), lambda i: (i, 0)),
            pl.BlockSpec(W_e1.shape, lambda i: (0, 0)),
            pl.BlockSpec((1, HID), lambda i: (0, 0)),
            pl.BlockSpec(W_e2.shape, lambda i: (0, 0)),
            pl.BlockSpec((1, HID), lambda i: (0, 0)),
            pl.BlockSpec(W_h1.shape, lambda i: (0, 0)),
            pl.BlockSpec((1, HID), lambda i: (0, 0)),
            pl.BlockSpec(W_h2.shape, lambda i: (0, 0)),
            pl.BlockSpec((1, 1), lambda i: (0, 0)),
        ],
        out_specs=pl.BlockSpec((BB, n), lambda i: (i, 0)),
        out_shape=jax.ShapeDtypeStruct((B, n), jnp.float32),
    )(type_token.astype(jnp.int32), v56, W_e1, b_e1.reshape(1, -1),
      W_e2, b_e2.reshape(1, -1), W_h1, b_h1.reshape(1, -1),
      W_h2, b_h2.reshape(1, 1))
    return out.reshape(1, B * n, 1)


# speedup vs baseline: 3.1942x; 2.9508x over previous
import jax
import jax.numpy as jnp
from jax.experimental import pallas as pl

HID = 64
NOBJ = 14
TOK = 9


def _dot(x, w):
    return jax.lax.dot_general(
        x, w, (((1,), (0,)), ((), ())), preferred_element_type=jnp.float32,
        precision=jax.lax.Precision.DEFAULT,
    )


def _bd(w):
    z = jnp.zeros_like(w)
    return jnp.concatenate(
        [jnp.concatenate([w, z], axis=1), jnp.concatenate([z, w], axis=1)],
        axis=0)


def _mp_kernel(tt_ref, v_ref, we1_ref, be1_ref, we2_ref, be2_ref,
               wh1_ref, bh1_ref, wh2_ref, bh2_ref, out_ref):
    f32 = jnp.float32
    tt = tt_ref[0, :]
    oh = (jax.lax.broadcasted_iota(jnp.int32, (NOBJ, TOK), 1)
          == tt[:, None]).astype(f32)
    we1 = we1_ref[:]
    Si = oh @ we1[0:TOK, :]
    Sj = oh @ we1[TOK:2 * TOK, :]
    WA = we1[19:23, :] + we1[51:55, :]
    WB = we1[35:39, :] - we1[51:55, :]
    wh1 = wh1_ref[:]
    Tn = oh @ wh1[0:TOK, :]
    Wm = wh1[TOK:TOK + HID, :]
    WV = wh1[TOK + HID + 1:TOK + HID + 5, :]
    be1 = be1_ref[:]
    be2 = be2_ref[:]
    bh1 = bh1_ref[:]
    bh2 = bh2_ref[:]
    we2 = we2_ref[:]
    wh2 = wh2_ref[:]

    WA2 = _bd(WA)
    WB2 = jnp.concatenate([WB, WB], axis=1)
    Wem2 = _bd(_dot(we2, Wm))
    WV2 = _bd(WV)
    Wh2_2 = _bd(wh2)
    Sib = Si + be1
    Tnb = Tn + bh1 + float(NOBJ) * _dot(be2, Wm)

    v = v_ref[:]
    Bs = [_dot(v[:, 4 * c:4 * (c + 1)], WB2)
          + jnp.concatenate([Sj[c:c + 1, :]] * 2, axis=1)
          for c in range(NOBJ)]
    cols = []
    for p in range(NOBJ // 2):
        a0, a1 = 2 * p, 2 * p + 1
        vs2 = v[:, 8 * p:8 * p + 8]
        row = jnp.concatenate([Sib[a0:a0 + 1, :], Sib[a1:a1 + 1, :]], axis=1)
        A2 = _dot(vs2, WA2) + row
        terms = [jnp.maximum(A2 + Bs[c], 0.0) for c in range(NOBJ)]
        while len(terms) > 1:
            terms = [terms[i] + terms[i + 1] for i in range(0, len(terms) - 1, 2)] \
                + ([terms[-1]] if len(terms) % 2 else [])
        acc = terms[0]
        rowt = jnp.concatenate([Tnb[a0:a0 + 1, :], Tnb[a1:a1 + 1, :]], axis=1)
        z = jnp.maximum(_dot(acc, Wem2) + _dot(vs2, WV2) + rowt, 0.0)
        cols.append(_dot(z, Wh2_2) + bh2)
    out_ref[:, :] = jnp.concatenate(cols, axis=1)


def kernel(inputs, type_token, global_token, W_e1, b_e1, W_e2, b_e2,
           W_h1, b_h1, W_h2, b_h2):
    nproc, B, n, d = inputs.shape
    v56 = inputs.reshape(B, n * d)
    BB = 1024
    out = pl.pallas_call(
        _mp_kernel,
        grid=(B // BB,),
        in_specs=[
            pl.BlockSpec((1, n), lambda i: (0, 0)),
            pl.BlockSpec((BB, n * d), lambda i: (i, 0)),
            pl.BlockSpec(W_e1.shape, lambda i: (0, 0)),
            pl.BlockSpec((1, HID), lambda i: (0, 0)),
            pl.BlockSpec(W_e2.shape, lambda i: (0, 0)),
            pl.BlockSpec((1, HID), lambda i: (0, 0)),
            pl.BlockSpec(W_h1.shape, lambda i: (0, 0)),
            pl.BlockSpec((1, HID), lambda i: (0, 0)),
            pl.BlockSpec(W_h2.shape, lambda i: (0, 0)),
            pl.BlockSpec((1, 1), lambda i: (0, 0)),
        ],
        out_specs=pl.BlockSpec((BB, n), lambda i: (i, 0)),
        out_shape=jax.ShapeDtypeStruct((B, n), jnp.float32),
    )(type_token.astype(jnp.int32), v56, W_e1, b_e1.reshape(1, -1),
      W_e2, b_e2.reshape(1, -1), W_h1, b_h1.reshape(1, -1),
      W_h2, b_h2.reshape(1, 1))
    return out.reshape(1, B * n, 1)


# speedup vs baseline: 4.0209x; 1.2588x over previous
import jax
import jax.numpy as jnp
from jax.experimental import pallas as pl

HID = 64
NOBJ = 14
TOK = 9


def _dot(x, w):
    return jax.lax.dot_general(
        x, w, (((1,), (0,)), ((), ())), preferred_element_type=jnp.float32,
        precision=jax.lax.Precision.DEFAULT,
    )


def _bd(w):
    z = jnp.zeros_like(w)
    return jnp.concatenate(
        [jnp.concatenate([w, z], axis=1), jnp.concatenate([z, w], axis=1)],
        axis=0)


def _mp_kernel(tt_ref, v_ref, we1_ref, be1_ref, we2_ref, be2_ref,
               wh1_ref, bh1_ref, wh2_ref, bh2_ref, out_ref):
    f32 = jnp.float32
    tt = tt_ref[0, :]
    oh = (jax.lax.broadcasted_iota(jnp.int32, (NOBJ, TOK), 1)
          == tt[:, None]).astype(f32)
    we1 = we1_ref[:]
    Si = oh @ we1[0:TOK, :]
    Sj = oh @ we1[TOK:2 * TOK, :]
    WA = we1[19:23, :] + we1[51:55, :]
    WB = we1[35:39, :] - we1[51:55, :]
    wh1 = wh1_ref[:]
    Tn = oh @ wh1[0:TOK, :]
    Wm = wh1[TOK:TOK + HID, :]
    WV = wh1[TOK + HID + 1:TOK + HID + 5, :]
    be1 = be1_ref[:]
    be2 = be2_ref[:]
    bh1 = bh1_ref[:]
    bh2 = bh2_ref[:]
    we2 = we2_ref[:]
    wh2 = wh2_ref[:]

    WA2 = _bd(WA)
    WB2 = jnp.concatenate([WB, WB], axis=1)
    Wem2 = _bd(_dot(we2, Wm))
    WV2 = _bd(WV)
    Wh2_2 = _bd(wh2)
    Sib = Si + be1
    Tnb = Tn + bh1 + float(NOBJ) * _dot(be2, Wm)

    NP = NOBJ // 2
    zcol = jnp.zeros((2 * HID, 2), dtype=f32)
    Wh2_14 = jnp.concatenate(
        [jnp.concatenate([zcol] * p + [Wh2_2] + [zcol] * (NP - 1 - p), axis=0)
         for p in range(NP)], axis=1)

    bf = jnp.bfloat16
    Wem2h = Wem2.astype(bf)
    v = v_ref[:]
    Bs = [(_dot(v[:, 4 * c:4 * (c + 1)], WB2)
           + jnp.concatenate([Sj[c:c + 1, :]] * 2, axis=1)).astype(bf)
          for c in range(NOBJ)]
    zs = []
    for p in range(NP):
        a0, a1 = 2 * p, 2 * p + 1
        vs2 = v[:, 8 * p:8 * p + 8]
        row = jnp.concatenate([Sib[a0:a0 + 1, :], Sib[a1:a1 + 1, :]], axis=1)
        A2 = (_dot(vs2, WA2) + row).astype(bf)
        terms = [jnp.maximum(A2 + Bs[c], 0) for c in range(NOBJ)]
        while len(terms) > 1:
            terms = [terms[i] + terms[i + 1] for i in range(0, len(terms) - 1, 2)] \
                + ([terms[-1]] if len(terms) % 2 else [])
        acc = terms[0]
        rowt = jnp.concatenate([Tnb[a0:a0 + 1, :], Tnb[a1:a1 + 1, :]], axis=1)
        zs.append(jnp.maximum(_dot(acc, Wem2h) + _dot(vs2, WV2) + rowt, 0.0))
    zall = jnp.concatenate(zs, axis=1)
    out_ref[:, :] = _dot(zall, Wh2_14) + bh2


def kernel(inputs, type_token, global_token, W_e1, b_e1, W_e2, b_e2,
           W_h1, b_h1, W_h2, b_h2):
    nproc, B, n, d = inputs.shape
    v56 = inputs.reshape(B, n * d)
    BB = 1024
    out = pl.pallas_call(
        _mp_kernel,
        grid=(B // BB,),
        in_specs=[
            pl.BlockSpec((1, n), lambda i: (0, 0)),
            pl.BlockSpec((BB, n * d), lambda i: (i, 0)),
            pl.BlockSpec(W_e1.shape, lambda i: (0, 0)),
            pl.BlockSpec((1, HID), lambda i: (0, 0)),
            pl.BlockSpec(W_e2.shape, lambda i: (0, 0)),
            pl.BlockSpec((1, HID), lambda i: (0, 0)),
            pl.BlockSpec(W_h1.shape, lambda i: (0, 0)),
            pl.BlockSpec((1, HID), lambda i: (0, 0)),
            pl.BlockSpec(W_h2.shape, lambda i: (0, 0)),
            pl.BlockSpec((1, 1), lambda i: (0, 0)),
        ],
        out_specs=pl.BlockSpec((BB, n), lambda i: (i, 0)),
        out_shape=jax.ShapeDtypeStruct((B, n), jnp.float32),
    )(type_token.astype(jnp.int32), v56, W_e1, b_e1.reshape(1, -1),
      W_e2, b_e2.reshape(1, -1), W_h1, b_h1.reshape(1, -1),
      W_h2, b_h2.reshape(1, 1))
    return out.reshape(1, B * n, 1)


# speedup vs baseline: 4.0369x; 1.0040x over previous
import jax
import jax.numpy as jnp
from jax.experimental import pallas as pl
from jax.experimental.pallas import tpu as pltpu

HID = 64
NOBJ = 14
TOK = 9


def _dot(x, w):
    return jax.lax.dot_general(
        x, w, (((1,), (0,)), ((), ())), preferred_element_type=jnp.float32,
        precision=jax.lax.Precision.DEFAULT,
    )


def _bd(w):
    z = jnp.zeros_like(w)
    return jnp.concatenate(
        [jnp.concatenate([w, z], axis=1), jnp.concatenate([z, w], axis=1)],
        axis=0)


def _mp_kernel(tt_ref, v_ref, we1_ref, be1_ref, we2_ref, be2_ref,
               wh1_ref, bh1_ref, wh2_ref, bh2_ref, out_ref):
    f32 = jnp.float32
    tt = tt_ref[0, :]
    oh = (jax.lax.broadcasted_iota(jnp.int32, (NOBJ, TOK), 1)
          == tt[:, None]).astype(f32)
    we1 = we1_ref[:]
    Si = oh @ we1[0:TOK, :]
    Sj = oh @ we1[TOK:2 * TOK, :]
    WA = we1[19:23, :] + we1[51:55, :]
    WB = we1[35:39, :] - we1[51:55, :]
    wh1 = wh1_ref[:]
    Tn = oh @ wh1[0:TOK, :]
    Wm = wh1[TOK:TOK + HID, :]
    WV = wh1[TOK + HID + 1:TOK + HID + 5, :]
    be1 = be1_ref[:]
    be2 = be2_ref[:]
    bh1 = bh1_ref[:]
    bh2 = bh2_ref[:]
    we2 = we2_ref[:]
    wh2 = wh2_ref[:]

    WA2 = _bd(WA)
    WB2 = jnp.concatenate([WB, WB], axis=1)
    Wem2 = _bd(_dot(we2, Wm))
    WV2 = _bd(WV)
    Wh2_2 = _bd(wh2)
    Sib = Si + be1
    Tnb = Tn + bh1 + float(NOBJ) * _dot(be2, Wm)

    NP = NOBJ // 2
    zcol = jnp.zeros((2 * HID, 2), dtype=f32)
    Wh2_14 = jnp.concatenate(
        [jnp.concatenate([zcol] * p + [Wh2_2] + [zcol] * (NP - 1 - p), axis=0)
         for p in range(NP)], axis=1)

    bf = jnp.bfloat16
    Wem2h = Wem2.astype(bf)
    v = v_ref[:]
    Bs = [(_dot(v[:, 4 * c:4 * (c + 1)], WB2)
           + jnp.concatenate([Sj[c:c + 1, :]] * 2, axis=1)).astype(bf)
          for c in range(NOBJ)]
    zs = []
    for p in range(NP):
        a0, a1 = 2 * p, 2 * p + 1
        vs2 = v[:, 8 * p:8 * p + 8]
        row = jnp.concatenate([Sib[a0:a0 + 1, :], Sib[a1:a1 + 1, :]], axis=1)
        A2 = (_dot(vs2, WA2) + row).astype(bf)
        terms = [jnp.maximum(A2 + Bs[c], 0) for c in range(NOBJ)]
        while len(terms) > 1:
            terms = [terms[i] + terms[i + 1] for i in range(0, len(terms) - 1, 2)] \
                + ([terms[-1]] if len(terms) % 2 else [])
        acc = terms[0]
        rowt = jnp.concatenate([Tnb[a0:a0 + 1, :], Tnb[a1:a1 + 1, :]], axis=1)
        zs.append(jnp.maximum(_dot(acc, Wem2h) + _dot(vs2, WV2) + rowt, 0.0))
    zall = jnp.concatenate(zs, axis=1)
    out_ref[:, :] = _dot(zall, Wh2_14) + bh2


def kernel(inputs, type_token, global_token, W_e1, b_e1, W_e2, b_e2,
           W_h1, b_h1, W_h2, b_h2):
    nproc, B, n, d = inputs.shape
    v56 = inputs.reshape(B, n * d)
    BB = 1024
    out = pl.pallas_call(
        _mp_kernel,
        grid=(B // BB,),
        in_specs=[
            pl.BlockSpec((1, n), lambda i: (0, 0)),
            pl.BlockSpec((BB, n * d), lambda i: (i, 0)),
            pl.BlockSpec(W_e1.shape, lambda i: (0, 0)),
            pl.BlockSpec((1, HID), lambda i: (0, 0)),
            pl.BlockSpec(W_e2.shape, lambda i: (0, 0)),
            pl.BlockSpec((1, HID), lambda i: (0, 0)),
            pl.BlockSpec(W_h1.shape, lambda i: (0, 0)),
            pl.BlockSpec((1, HID), lambda i: (0, 0)),
            pl.BlockSpec(W_h2.shape, lambda i: (0, 0)),
            pl.BlockSpec((1, 1), lambda i: (0, 0)),
        ],
        out_specs=pl.BlockSpec((BB, n), lambda i: (i, 0)),
        out_shape=jax.ShapeDtypeStruct((B, n), jnp.float32),
        compiler_params=pltpu.CompilerParams(
            dimension_semantics=("parallel",)),
    )(type_token.astype(jnp.int32), v56, W_e1, b_e1.reshape(1, -1),
      W_e2, b_e2.reshape(1, -1), W_h1, b_h1.reshape(1, -1),
      W_h2, b_h2.reshape(1, 1))
    return out.reshape(1, B * n, 1)
